# Initial kernel scaffold; baseline (speedup 1.0000x reference)
import jax
import jax.numpy as jnp
from jax.experimental import pallas as pl


def kernel(feat, edge_index, W_src, b_src, W_dst, b_dst, attn):
    raise NotImplementedError("write your pallas kernel here")


# speedup vs baseline: 8.3019x; 8.3019x over previous
import functools

import jax
import jax.numpy as jnp
from jax import lax
from jax.experimental import pallas as pl
from jax.experimental.pallas import tpu as pltpu
from jax.experimental.pallas import tpu_sc as plsc

N = 10000
E = 320000
D = 128
NP = 10240

NC = 2
NS = 16
NW = NC * NS
EPW = E // NW
C = 80
NCHUNK = EPW // C
RPT = NP // NS
MMB = 1000

_mesh = plsc.VectorSubcoreMesh(core_axis_name="c", subcore_axis_name="s")


def _mm_body(feat_ref, ws_ref, bs_ref, wd_ref, bd_ref, fs_ref, fd_ref):
    x = feat_ref[...]
    dn = (((1,), (1,)), ((), ()))
    fs_ref[...] = lax.dot_general(x, ws_ref[...], dn,
                                  preferred_element_type=jnp.float32) + bs_ref[...]
    fd_ref[...] = lax.dot_general(x, wd_ref[...], dn,
                                  preferred_element_type=jnp.float32) + bd_ref[...]


def _mm(feat, W_src, b_src2, W_dst, b_dst2):
    return pl.pallas_call(
        _mm_body,
        grid=(N // MMB,),
        in_specs=[
            pl.BlockSpec((MMB, D), lambda i: (i, 0)),
            pl.BlockSpec((D, D), lambda i: (0, 0)),
            pl.BlockSpec((1, D), lambda i: (0, 0)),
            pl.BlockSpec((D, D), lambda i: (0, 0)),
            pl.BlockSpec((1, D), lambda i: (0, 0)),
        ],
        out_specs=[
            pl.BlockSpec((MMB, D), lambda i: (i, 0)),
            pl.BlockSpec((MMB, D), lambda i: (i, 0)),
        ],
        out_shape=[
            jax.ShapeDtypeStruct((N, D), jnp.float32),
            jax.ShapeDtypeStruct((N, D), jnp.float32),
        ],
    )(feat, W_src, b_src2, W_dst, b_dst2)


@functools.partial(
    pl.kernel,
    out_type=[
        jax.ShapeDtypeStruct((E,), jnp.float32),
        jax.ShapeDtypeStruct((NC, NP), jnp.float32),
    ],
    mesh=_mesh,
    scratch_types=[
        pltpu.VMEM((C,), jnp.int32),
        pltpu.VMEM((C,), jnp.int32),
        pltpu.VMEM((C, D), jnp.float32),
        pltpu.VMEM((C, D), jnp.float32),
        pltpu.VMEM((C,), jnp.float32),
        pltpu.VMEM((D,), jnp.float32),
        pltpu.VMEM((D,), jnp.float32),
        pltpu.VMEM((RPT,), jnp.float32),
        pltpu.VMEM_SHARED((NP,), jnp.float32),
        pltpu.SemaphoreType.DMA,
        pltpu.SemaphoreType.DMA,
    ],
    compiler_params=pltpu.CompilerParams(needs_layout_passes=False),
)
def _scores(fs_hbm, fd_hbm, src_hbm, dst_hbm, a06_hbm, a04_hbm,
            exp_hbm, denp_hbm,
            src_v, dst_v, fs_rows, fd_rows, exp_v, a06_v, a04_v, zslice,
            den_sh, sem1, sem2):
    cid = lax.axis_index("c")
    sid = lax.axis_index("s")
    wid = sid * NC + cid
    base0 = wid * EPW

    pltpu.sync_copy(a06_hbm, a06_v)
    pltpu.sync_copy(a04_hbm, a04_v)

    zero16 = jnp.zeros((16,), jnp.float32)
    lanes = lax.iota(jnp.int32, 16)

    for i in range(RPT // 16):
        zslice[pl.ds(i * 16, 16)] = zero16
    pltpu.sync_copy(zslice, den_sh.at[pl.ds(sid * RPT, RPT)])
    plsc.subcore_barrier()

    def _chunk(ci, carry):
        base = base0 + ci * C
        pltpu.sync_copy(src_hbm.at[pl.ds(base, C)], src_v)
        pltpu.sync_copy(dst_hbm.at[pl.ds(base, C)], dst_v)
        cp1 = pltpu.async_copy(fs_hbm.at[src_v], fs_rows, sem1)
        cp2 = pltpu.async_copy(fd_hbm.at[dst_v], fd_rows, sem2)
        cp1.wait()
        cp2.wait()
        a06c = [a06_v[pl.ds(j * 16, 16)] for j in range(D // 16)]
        a04c = [a04_v[pl.ds(j * 16, 16)] for j in range(D // 16)]

        def _group(g, carry2):
            score_vec = zero16
            for l in range(16):
                e = g * 16 + l
                acc = zero16
                for j in range(D // 16):
                    a = (fs_rows[e, pl.ds(j * 16, 16)]
                         + fd_rows[e, pl.ds(j * 16, 16)])
                    acc = acc + a06c[j] * a + a04c[j] * jnp.abs(a)
                s = jnp.sum(acc)
                score_vec = jnp.where(lanes == l, jnp.full((16,), s, jnp.float32),
                                      score_vec)
            exp_v[pl.ds(g * 16, 16)] = jnp.exp(score_vec)
            return carry2

        lax.fori_loop(0, C // 16, _group, 0)
        pltpu.sync_copy(exp_v, exp_hbm.at[pl.ds(base, C)])
        pltpu.sync_copy(exp_v, den_sh.at[dst_v], add=True)
        return carry

    lax.fori_loop(0, NCHUNK, _chunk, 0)
    plsc.subcore_barrier()

    @pl.when(sid == 0)
    def _copy_out():
        pltpu.sync_copy(den_sh, denp_hbm.at[cid])


@functools.partial(
    pl.kernel,
    out_type=jax.ShapeDtypeStruct((NC, N, D), jnp.float32),
    mesh=_mesh,
    scratch_types=[
        pltpu.VMEM((C,), jnp.int32),
        pltpu.VMEM((C,), jnp.int32),
        pltpu.VMEM((C,), jnp.float32),
        pltpu.VMEM((C, D), jnp.float32),
        pltpu.VMEM((32, D), jnp.float32),
        pltpu.VMEM((NC, NP), jnp.float32),
        pltpu.VMEM((NP,), jnp.float32),
        pltpu.VMEM_SHARED((NP, D), jnp.float32),
        pltpu.SemaphoreType.DMA,
    ],
    compiler_params=pltpu.CompilerParams(needs_layout_passes=False),
)
def _agg(fs_hbm, src_hbm, dst_hbm, exp_hbm, denp_hbm,
         rstp_hbm,
         src_v, dst_v, exp_v, rows, zblk, dpart, den_v,
         rst_sh, sem):
    cid = lax.axis_index("c")
    sid = lax.axis_index("s")
    wid = sid * NC + cid
    base0 = wid * EPW

    zero16 = jnp.zeros((16,), jnp.float32)

    def _zb(i, carry):
        zblk[i // 8, pl.ds((i % 8) * 16, 16)] = zero16
        return carry

    lax.fori_loop(0, 32 * (D // 16), _zb, 0)
    for r in range(RPT // 32):
        pltpu.sync_copy(zblk, rst_sh.at[pl.ds(sid * RPT + r * 32, 32)])

    pltpu.sync_copy(denp_hbm, dpart)

    def _dmerge(i, carry):
        den_v[pl.ds(i * 16, 16)] = (dpart[0, pl.ds(i * 16, 16)]
                                    + dpart[1, pl.ds(i * 16, 16)])
        return carry

    lax.fori_loop(0, NP // 16, _dmerge, 0)
    plsc.subcore_barrier()

    def _chunk(ci, carry):
        base = base0 + ci * C
        pltpu.sync_copy(src_hbm.at[pl.ds(base, C)], src_v)
        pltpu.sync_copy(dst_hbm.at[pl.ds(base, C)], dst_v)
        pltpu.sync_copy(exp_hbm.at[pl.ds(base, C)], exp_v)
        cp = pltpu.async_copy(fs_hbm.at[src_v], rows, sem)
        cp.wait()
        for g in range(C // 16):
            dl = dst_v[pl.ds(g * 16, 16)]
            den_g = plsc.load_gather(den_v, [dl])
            a_g = exp_v[pl.ds(g * 16, 16)] / den_g
            for l in range(16):
                e = g * 16 + l
                af = jnp.full((16,), a_g[l], jnp.float32)
                for j in range(D // 16):
                    rows[e, pl.ds(j * 16, 16)] = rows[e, pl.ds(j * 16, 16)] * af
        pltpu.sync_copy(rows, rst_sh.at[dst_v], add=True)
        return carry

    lax.fori_loop(0, NCHUNK, _chunk, 0)
    plsc.subcore_barrier()

    @pl.when(sid < NS - 1)
    def _copy_full():
        pltpu.sync_copy(rst_sh.at[pl.ds(sid * RPT, RPT)],
                        rstp_hbm.at[cid, pl.ds(sid * RPT, RPT)])

    @pl.when(sid == NS - 1)
    def _copy_tail():
        tail = N - (NS - 1) * RPT
        pltpu.sync_copy(rst_sh.at[pl.ds((NS - 1) * RPT, tail)],
                        rstp_hbm.at[cid, pl.ds((NS - 1) * RPT, tail)])


def _add_body(p_ref, o_ref):
    o_ref[...] = p_ref[0] + p_ref[1]


def _final_add(rstp):
    return pl.pallas_call(
        _add_body,
        grid=(N // MMB,),
        in_specs=[pl.BlockSpec((NC, MMB, D), lambda i: (0, i, 0))],
        out_specs=pl.BlockSpec((MMB, D), lambda i: (i, 0)),
        out_shape=jax.ShapeDtypeStruct((N, D), jnp.float32),
    )(rstp)


def kernel(feat, edge_index, W_src, b_src, W_dst, b_dst, attn):
    src = edge_index[0].astype(jnp.int32)
    dst = edge_index[1].astype(jnp.int32)
    fs, fd = _mm(feat, W_src, b_src.reshape(1, D), W_dst, b_dst.reshape(1, D))
    a06 = 0.6 * attn
    a04 = 0.4 * attn
    expsc, denp = _scores(fs, fd, src, dst, a06, a04)
    rstp = _agg(fs, src, dst, expsc, denp)
    return _final_add(rstp)


# speedup vs baseline: 11.7040x; 1.4098x over previous
import functools

import jax
import jax.numpy as jnp
from jax import lax
from jax.experimental import pallas as pl
from jax.experimental.pallas import tpu as pltpu
from jax.experimental.pallas import tpu_sc as plsc

N = 10000
E = 320000
D = 128
NP = 10240

NC = 2
NS = 16
NW = NC * NS
EPW = E // NW
C = 80
NCHUNK = EPW // C
RPT = NP // NS
MMB = 1000

_mesh = plsc.VectorSubcoreMesh(core_axis_name="c", subcore_axis_name="s")
_sc_params = pltpu.CompilerParams(needs_layout_passes=False)


def _mm_body(feat_ref, ws_ref, bs_ref, wd_ref, bd_ref, fs_ref, fd_ref):
    x = feat_ref[...]
    dn = (((1,), (1,)), ((), ()))
    fs_ref[...] = lax.dot_general(x, ws_ref[...], dn,
                                  preferred_element_type=jnp.float32) + bs_ref[...]
    fd_ref[...] = lax.dot_general(x, wd_ref[...], dn,
                                  preferred_element_type=jnp.float32) + bd_ref[...]


def _mm(feat, W_src, b_src2, W_dst, b_dst2):
    return pl.pallas_call(
        _mm_body,
        grid=(N // MMB,),
        in_specs=[
            pl.BlockSpec((MMB, D), lambda i: (i, 0)),
            pl.BlockSpec((D, D), lambda i: (0, 0)),
            pl.BlockSpec((1, D), lambda i: (0, 0)),
            pl.BlockSpec((D, D), lambda i: (0, 0)),
            pl.BlockSpec((1, D), lambda i: (0, 0)),
        ],
        out_specs=[
            pl.BlockSpec((MMB, D), lambda i: (i, 0)),
            pl.BlockSpec((MMB, D), lambda i: (i, 0)),
        ],
        out_shape=[
            jax.ShapeDtypeStruct((N, D), jnp.float32),
            jax.ShapeDtypeStruct((N, D), jnp.float32),
        ],
    )(feat, W_src, b_src2, W_dst, b_dst2)


@functools.partial(
    pl.kernel,
    out_type=[
        jax.ShapeDtypeStruct((E,), jnp.float32),
        jax.ShapeDtypeStruct((NC, NP), jnp.float32),
    ],
    mesh=_mesh,
    scratch_types=(
        [pltpu.VMEM((C,), jnp.int32)] * 2
        + [pltpu.VMEM((C,), jnp.int32)] * 2
        + [pltpu.VMEM((C, D), jnp.float32)] * 2
        + [pltpu.VMEM((C, D), jnp.float32)] * 2
        + [pltpu.VMEM((C,), jnp.float32)] * 2
        + [pltpu.VMEM((D,), jnp.float32)] * 2
        + [pltpu.VMEM((RPT,), jnp.float32)]
        + [pltpu.VMEM_SHARED((NP,), jnp.float32)]
        + [pltpu.SemaphoreType.DMA] * 6
    ),
    compiler_params=_sc_params,
)
def _scores(fs_hbm, fd_hbm, src_hbm, dst_hbm, a06_hbm, a04_hbm,
            exp_hbm, denp_hbm, *refs):
    src_v = refs[0:2]
    dst_v = refs[2:4]
    fs_rows = refs[4:6]
    fd_rows = refs[6:8]
    exp_v = refs[8:10]
    a06_v, a04_v = refs[10], refs[11]
    zslice = refs[12]
    den_sh = refs[13]
    sem_fs = refs[14:16]
    sem_fd = refs[16:18]
    sem_eo = refs[18:20]

    cid = lax.axis_index("c")
    sid = lax.axis_index("s")
    wid = sid * NC + cid
    base0 = wid * EPW

    pltpu.sync_copy(a06_hbm, a06_v)
    pltpu.sync_copy(a04_hbm, a04_v)

    zero16 = jnp.zeros((16,), jnp.float32)
    lanes = lax.iota(jnp.int32, 16)

    for i in range(RPT // 16):
        zslice[pl.ds(i * 16, 16)] = zero16
    pltpu.sync_copy(zslice, den_sh.at[pl.ds(sid * RPT, RPT)])
    plsc.subcore_barrier()

    a06c = [a06_v[pl.ds(j * 16, 16)] for j in range(D // 16)]
    a04c = [a04_v[pl.ds(j * 16, 16)] for j in range(D // 16)]

    def _issue(ci, s):
        base = base0 + ci * C
        pltpu.sync_copy(src_hbm.at[pl.ds(base, C)], src_v[s])
        pltpu.sync_copy(dst_hbm.at[pl.ds(base, C)], dst_v[s])
        pltpu.async_copy(fs_hbm.at[src_v[s]], fs_rows[s], sem_fs[s])
        pltpu.async_copy(fd_hbm.at[dst_v[s]], fd_rows[s], sem_fd[s])

    def _wait_g(s):
        pltpu.make_async_copy(fs_hbm.at[pl.ds(0, C)], fs_rows[s], sem_fs[s]).wait()
        pltpu.make_async_copy(fs_hbm.at[pl.ds(0, C)], fd_rows[s], sem_fd[s]).wait()

    def _wait_eo(s):
        pltpu.make_async_copy(exp_hbm.at[pl.ds(0, C)], exp_v[s], sem_eo[s]).wait()

    def _compute(ci, s):
        def _group(g, carry):
            score_vec = zero16
            for l in range(16):
                e = g * 16 + l
                acc = zero16
                for j in range(D // 16):
                    a = (fs_rows[s][e, pl.ds(j * 16, 16)]
                         + fd_rows[s][e, pl.ds(j * 16, 16)])
                    acc = acc + a06c[j] * a + a04c[j] * jnp.abs(a)
                sc = jnp.sum(acc)
                score_vec = jnp.where(lanes == l,
                                      jnp.full((16,), sc, jnp.float32), score_vec)
            exp_v[s][pl.ds(g * 16, 16)] = jnp.exp(score_vec)
            return carry

        lax.fori_loop(0, C // 16, _group, 0)
        pltpu.async_copy(exp_v[s], exp_hbm.at[pl.ds(base0 + ci * C, C)], sem_eo[s])
        pltpu.sync_copy(exp_v[s], den_sh.at[dst_v[s]], add=True)

    _issue(0, 0)
    _issue(1, 1)
    _wait_g(0)
    _compute(0, 0)
    _issue(2, 0)
    _wait_g(1)
    _compute(1, 1)

    def _pair(i, carry):
        ci0 = 2 * i
        _issue(ci0 + 1, 1)
        _wait_g(0)
        _wait_eo(0)
        _compute(ci0, 0)
        _issue(ci0 + 2, 0)
        _wait_g(1)
        _wait_eo(1)
        _compute(ci0 + 1, 1)
        return carry

    lax.fori_loop(1, NCHUNK // 2, _pair, 0)

    _wait_g(0)
    _wait_eo(0)
    _compute(NCHUNK - 1, 0)
    _wait_eo(1)
    _wait_eo(0)
    plsc.subcore_barrier()

    @pl.when(sid == 0)
    def _copy_out():
        pltpu.sync_copy(den_sh, denp_hbm.at[cid])


def _dmerge_body(p_ref, o_ref):
    o_ref[...] = p_ref[0] + p_ref[1]


def _den_merge(denp):
    return pl.pallas_call(
        _dmerge_body,
        in_specs=[pl.BlockSpec((NC, NP // D, D), lambda: (0, 0, 0))],
        out_specs=pl.BlockSpec((NP // D, D), lambda: (0, 0)),
        out_shape=jax.ShapeDtypeStruct((NP // D, D), jnp.float32),
    )(denp.reshape(NC, NP // D, D)).reshape(NP)


@functools.partial(
    pl.kernel,
    out_type=jax.ShapeDtypeStruct((NC, N, D), jnp.float32),
    mesh=_mesh,
    scratch_types=(
        [pltpu.VMEM((C,), jnp.int32)] * 2
        + [pltpu.VMEM((C,), jnp.int32)] * 2
        + [pltpu.VMEM((C,), jnp.float32)] * 2
        + [pltpu.VMEM((C, D), jnp.float32)] * 2
        + [pltpu.VMEM((C, D), jnp.float32)]
        + [pltpu.VMEM((NP,), jnp.float32)]
        + [pltpu.VMEM_SHARED((NP, D), jnp.float32)]
        + [pltpu.SemaphoreType.DMA] * 2
    ),
    compiler_params=_sc_params,
)
def _agg(fs_hbm, src_hbm, dst_hbm, exp_hbm, den_hbm,
         rstp_hbm, *refs):
    src_v = refs[0:2]
    dst_v = refs[2:4]
    exp_v = refs[4:6]
    rows = refs[6:8]
    zblk = refs[8]
    den_v = refs[9]
    rst_sh = refs[10]
    sem_r = refs[11:13]

    cid = lax.axis_index("c")
    sid = lax.axis_index("s")
    wid = sid * NC + cid
    base0 = wid * EPW

    zero16 = jnp.zeros((16,), jnp.float32)

    def _zb(i, carry):
        zblk[i // 8, pl.ds((i % 8) * 16, 16)] = zero16
        return carry

    lax.fori_loop(0, C * (D // 16), _zb, 0)
    for r in range(RPT // C):
        pltpu.sync_copy(zblk, rst_sh.at[pl.ds(sid * RPT + r * C, C)])

    pltpu.sync_copy(den_hbm, den_v)
    plsc.subcore_barrier()

    def _issue(ci, s):
        base = base0 + ci * C
        pltpu.sync_copy(src_hbm.at[pl.ds(base, C)], src_v[s])
        pltpu.sync_copy(dst_hbm.at[pl.ds(base, C)], dst_v[s])
        pltpu.sync_copy(exp_hbm.at[pl.ds(base, C)], exp_v[s])
        pltpu.async_copy(fs_hbm.at[src_v[s]], rows[s], sem_r[s])

    def _wait_g(s):
        pltpu.make_async_copy(fs_hbm.at[pl.ds(0, C)], rows[s], sem_r[s]).wait()

    def _compute(ci, s):
        for g in range(C // 16):
            dl = dst_v[s][pl.ds(g * 16, 16)]
            den_g = plsc.load_gather(den_v, [dl])
            a_g = exp_v[s][pl.ds(g * 16, 16)] / den_g
            for l in range(16):
                e = g * 16 + l
                af = jnp.full((16,), a_g[l], jnp.float32)
                for j in range(D // 16):
                    rows[s][e, pl.ds(j * 16, 16)] = (
                        rows[s][e, pl.ds(j * 16, 16)] * af)
        pltpu.sync_copy(rows[s], rst_sh.at[dst_v[s]], add=True)

    _issue(0, 0)
    _issue(1, 1)
    _wait_g(0)
    _compute(0, 0)
    _issue(2, 0)
    _wait_g(1)
    _compute(1, 1)

    def _pair(i, carry):
        ci0 = 2 * i
        _issue(ci0 + 1, 1)
        _wait_g(0)
        _compute(ci0, 0)
        _issue(ci0 + 2, 0)
        _wait_g(1)
        _compute(ci0 + 1, 1)
        return carry

    lax.fori_loop(1, NCHUNK // 2, _pair, 0)

    _wait_g(0)
    _compute(NCHUNK - 1, 0)
    plsc.subcore_barrier()

    @pl.when(sid < NS - 1)
    def _copy_full():
        pltpu.sync_copy(rst_sh.at[pl.ds(sid * RPT, RPT)],
                        rstp_hbm.at[cid, pl.ds(sid * RPT, RPT)])

    @pl.when(sid == NS - 1)
    def _copy_tail():
        tail = N - (NS - 1) * RPT
        pltpu.sync_copy(rst_sh.at[pl.ds((NS - 1) * RPT, tail)],
                        rstp_hbm.at[cid, pl.ds((NS - 1) * RPT, tail)])


def _add_body(p_ref, o_ref):
    o_ref[...] = p_ref[0] + p_ref[1]


def _final_add(rstp):
    return pl.pallas_call(
        _add_body,
        grid=(N // MMB,),
        in_specs=[pl.BlockSpec((NC, MMB, D), lambda i: (0, i, 0))],
        out_specs=pl.BlockSpec((MMB, D), lambda i: (i, 0)),
        out_shape=jax.ShapeDtypeStruct((N, D), jnp.float32),
    )(rstp)


def kernel(feat, edge_index, W_src, b_src, W_dst, b_dst, attn):
    src = edge_index[0].astype(jnp.int32)
    dst = edge_index[1].astype(jnp.int32)
    fs, fd = _mm(feat, W_src, b_src.reshape(1, D), W_dst, b_dst.reshape(1, D))
    a06 = 0.6 * attn
    a04 = 0.4 * attn
    expsc, denp = _scores(fs, fd, src, dst, a06, a04)
    den = _den_merge(denp)
    rstp = _agg(fs, src, dst, expsc, den)
    return _final_add(rstp)


# speedup vs baseline: 13.9437x; 1.1914x over previous
import functools

import jax
import jax.numpy as jnp
from jax import lax
from jax.experimental import pallas as pl
from jax.experimental.pallas import tpu as pltpu
from jax.experimental.pallas import tpu_sc as plsc

N = 10000
E = 320000
D = 128
NP = 10240

NC = 2
NS = 16
NW = NC * NS
EPW = E // NW
C = 80
NCHUNK = EPW // C
RPT = NP // NS
MMB = 1000

_mesh = plsc.VectorSubcoreMesh(core_axis_name="c", subcore_axis_name="s")
_sc_params = pltpu.CompilerParams(needs_layout_passes=False)


def _mm_body(feat_ref, ws_ref, bs_ref, wd_ref, bd_ref, fs_ref, fd_ref):
    x = feat_ref[...]
    dn = (((1,), (1,)), ((), ()))
    fs_ref[...] = lax.dot_general(x, ws_ref[...], dn,
                                  preferred_element_type=jnp.float32) + bs_ref[...]
    fd_ref[...] = lax.dot_general(x, wd_ref[...], dn,
                                  preferred_element_type=jnp.float32) + bd_ref[...]


def _mm(feat, W_src, b_src2, W_dst, b_dst2):
    return pl.pallas_call(
        _mm_body,
        grid=(N // MMB,),
        in_specs=[
            pl.BlockSpec((MMB, D), lambda i: (i, 0)),
            pl.BlockSpec((D, D), lambda i: (0, 0)),
            pl.BlockSpec((1, D), lambda i: (0, 0)),
            pl.BlockSpec((D, D), lambda i: (0, 0)),
            pl.BlockSpec((1, D), lambda i: (0, 0)),
        ],
        out_specs=[
            pl.BlockSpec((MMB, D), lambda i: (i, 0)),
            pl.BlockSpec((MMB, D), lambda i: (i, 0)),
        ],
        out_shape=[
            jax.ShapeDtypeStruct((N, D), jnp.float32),
            jax.ShapeDtypeStruct((N, D), jnp.float32),
        ],
    )(feat, W_src, b_src2, W_dst, b_dst2)


@functools.partial(
    pl.kernel,
    out_type=[
        jax.ShapeDtypeStruct((E,), jnp.float32),
        jax.ShapeDtypeStruct((NC, NP), jnp.float32),
    ],
    mesh=_mesh,
    scratch_types=(
        [pltpu.VMEM((2 * C,), jnp.int32)] * 2
        + [pltpu.VMEM((C, D), jnp.float32)] * 2
        + [pltpu.VMEM((C, D), jnp.float32)] * 2
        + [pltpu.VMEM((C,), jnp.float32)] * 2
        + [pltpu.VMEM((D,), jnp.float32)] * 2
        + [pltpu.VMEM((RPT,), jnp.float32)]
        + [pltpu.VMEM_SHARED((NP,), jnp.float32)]
        + [pltpu.SemaphoreType.DMA] * 6
    ),
    compiler_params=_sc_params,
)
def _scores(fs_hbm, fd_hbm, ei_hbm, a06_hbm, a04_hbm,
            exp_hbm, denp_hbm, *refs):
    idx2 = refs[0:2]
    fs_rows = refs[2:4]
    fd_rows = refs[4:6]
    exp_v = refs[6:8]
    a06_v, a04_v = refs[8], refs[9]
    zslice = refs[10]
    den_sh = refs[11]
    sem_fs = refs[12:14]
    sem_fd = refs[14:16]
    sem_eo = refs[16:18]

    cid = lax.axis_index("c")
    sid = lax.axis_index("s")
    wid = sid * NC + cid
    base0 = wid * EPW

    pltpu.sync_copy(a06_hbm, a06_v)
    pltpu.sync_copy(a04_hbm, a04_v)

    zero16 = jnp.zeros((16,), jnp.float32)
    lanes = lax.iota(jnp.int32, 16)

    for i in range(RPT // 16):
        zslice[pl.ds(i * 16, 16)] = zero16
    pltpu.sync_copy(zslice, den_sh.at[pl.ds(sid * RPT, RPT)])
    plsc.subcore_barrier()

    a06c = [a06_v[pl.ds(j * 16, 16)] for j in range(D // 16)]
    a04c = [a04_v[pl.ds(j * 16, 16)] for j in range(D // 16)]

    def _issue(ci, s):
        base = base0 + ci * C
        pltpu.sync_copy(ei_hbm.at[pl.ds(2 * base, 2 * C)], idx2[s])
        pltpu.async_copy(fs_hbm.at[idx2[s].at[pl.ds(0, C)]], fs_rows[s], sem_fs[s])
        pltpu.async_copy(fd_hbm.at[idx2[s].at[pl.ds(C, C)]], fd_rows[s], sem_fd[s])

    def _wait_g(s):
        pltpu.make_async_copy(fs_hbm.at[pl.ds(0, C)], fs_rows[s], sem_fs[s]).wait()
        pltpu.make_async_copy(fs_hbm.at[pl.ds(0, C)], fd_rows[s], sem_fd[s]).wait()

    def _wait_eo(s):
        pltpu.make_async_copy(exp_hbm.at[pl.ds(0, C)], exp_v[s], sem_eo[s]).wait()

    def _compute(ci, s):
        def _group(g, carry):
            score_vec = zero16
            for l in range(16):
                e = g * 16 + l
                acc = zero16
                for j in range(D // 16):
                    a = (fs_rows[s][e, pl.ds(j * 16, 16)]
                         + fd_rows[s][e, pl.ds(j * 16, 16)])
                    acc = acc + a06c[j] * a + a04c[j] * jnp.abs(a)
                sc = jnp.sum(acc)
                score_vec = jnp.where(lanes == l,
                                      jnp.full((16,), sc, jnp.float32), score_vec)
            exp_v[s][pl.ds(g * 16, 16)] = jnp.exp(score_vec)
            return carry

        lax.fori_loop(0, C // 16, _group, 0)
        pltpu.async_copy(exp_v[s], exp_hbm.at[pl.ds(base0 + ci * C, C)], sem_eo[s])
        pltpu.sync_copy(exp_v[s], den_sh.at[idx2[s].at[pl.ds(C, C)]], add=True)

    _issue(0, 0)
    _issue(1, 1)
    _wait_g(0)
    _compute(0, 0)
    _issue(2, 0)
    _wait_g(1)
    _compute(1, 1)

    def _pair(i, carry):
        ci0 = 2 * i
        _issue(ci0 + 1, 1)
        _wait_g(0)
        _wait_eo(0)
        _compute(ci0, 0)
        _issue(ci0 + 2, 0)
        _wait_g(1)
        _wait_eo(1)
        _compute(ci0 + 1, 1)
        return carry

    lax.fori_loop(1, NCHUNK // 2, _pair, 0)

    _wait_g(0)
    _wait_eo(0)
    _compute(NCHUNK - 1, 0)
    _wait_eo(1)
    _wait_eo(0)
    plsc.subcore_barrier()

    @pl.when(sid == 0)
    def _copy_out():
        pltpu.sync_copy(den_sh, denp_hbm.at[cid])


def _dmerge_body(p_ref, o_ref):
    o_ref[...] = p_ref[0] + p_ref[1]


def _den_merge(denp):
    return pl.pallas_call(
        _dmerge_body,
        in_specs=[pl.BlockSpec((NC, NP // D, D), lambda: (0, 0, 0))],
        out_specs=pl.BlockSpec((NP // D, D), lambda: (0, 0)),
        out_shape=jax.ShapeDtypeStruct((NP // D, D), jnp.float32),
    )(denp.reshape(NC, NP // D, D)).reshape(NP)


@functools.partial(
    pl.kernel,
    out_type=jax.ShapeDtypeStruct((NC, N, D), jnp.float32),
    mesh=_mesh,
    scratch_types=(
        [pltpu.VMEM((2 * C,), jnp.int32)] * 2
        + [pltpu.VMEM((C,), jnp.float32)] * 2
        + [pltpu.VMEM((C, D), jnp.float32)] * 2
        + [pltpu.VMEM((C, D), jnp.float32)]
        + [pltpu.VMEM((NP,), jnp.float32)]
        + [pltpu.VMEM_SHARED((NP, D), jnp.float32)]
        + [pltpu.SemaphoreType.DMA] * 4
    ),
    compiler_params=_sc_params,
)
def _agg(fs_hbm, ei_hbm, exp_hbm, den_hbm,
         rstp_hbm, *refs):
    idx2 = refs[0:2]
    exp_v = refs[2:4]
    rows = refs[4:6]
    zblk = refs[6]
    den_v = refs[7]
    rst_sh = refs[8]
    sem_r = refs[9:11]
    sem_e = refs[11:13]

    cid = lax.axis_index("c")
    sid = lax.axis_index("s")
    wid = sid * NC + cid
    base0 = wid * EPW

    zero16 = jnp.zeros((16,), jnp.float32)

    def _zb(i, carry):
        zblk[i // 8, pl.ds((i % 8) * 16, 16)] = zero16
        return carry

    lax.fori_loop(0, C * (D // 16), _zb, 0)
    for r in range(RPT // C):
        pltpu.sync_copy(zblk, rst_sh.at[pl.ds(sid * RPT + r * C, C)])

    pltpu.sync_copy(den_hbm, den_v)
    plsc.subcore_barrier()

    def _issue(ci, s):
        base = base0 + ci * C
        pltpu.sync_copy(ei_hbm.at[pl.ds(2 * base, 2 * C)], idx2[s])
        pltpu.async_copy(exp_hbm.at[pl.ds(base, C)], exp_v[s], sem_e[s])
        pltpu.async_copy(fs_hbm.at[idx2[s].at[pl.ds(0, C)]], rows[s], sem_r[s])

    def _wait_g(s):
        pltpu.make_async_copy(fs_hbm.at[pl.ds(0, C)], rows[s], sem_r[s]).wait()
        pltpu.make_async_copy(exp_hbm.at[pl.ds(0, C)], exp_v[s], sem_e[s]).wait()

    def _compute(ci, s):
        for g in range(C // 16):
            dl = idx2[s][pl.ds(C + g * 16, 16)]
            den_g = plsc.load_gather(den_v, [dl])
            a_g = exp_v[s][pl.ds(g * 16, 16)] / den_g
            for l in range(16):
                e = g * 16 + l
                af = jnp.full((16,), a_g[l], jnp.float32)
                for j in range(D // 16):
                    rows[s][e, pl.ds(j * 16, 16)] = (
                        rows[s][e, pl.ds(j * 16, 16)] * af)
        pltpu.sync_copy(rows[s], rst_sh.at[idx2[s].at[pl.ds(C, C)]], add=True)

    _issue(0, 0)
    _issue(1, 1)
    _wait_g(0)
    _compute(0, 0)
    _issue(2, 0)
    _wait_g(1)
    _compute(1, 1)

    def _pair(i, carry):
        ci0 = 2 * i
        _issue(ci0 + 1, 1)
        _wait_g(0)
        _compute(ci0, 0)
        _issue(ci0 + 2, 0)
        _wait_g(1)
        _compute(ci0 + 1, 1)
        return carry

    lax.fori_loop(1, NCHUNK // 2, _pair, 0)

    _wait_g(0)
    _compute(NCHUNK - 1, 0)
    plsc.subcore_barrier()

    @pl.when(sid < NS - 1)
    def _copy_full():
        pltpu.sync_copy(rst_sh.at[pl.ds(sid * RPT, RPT)],
                        rstp_hbm.at[cid, pl.ds(sid * RPT, RPT)])

    @pl.when(sid == NS - 1)
    def _copy_tail():
        tail = N - (NS - 1) * RPT
        pltpu.sync_copy(rst_sh.at[pl.ds((NS - 1) * RPT, tail)],
                        rstp_hbm.at[cid, pl.ds((NS - 1) * RPT, tail)])


def _add_body(p_ref, o_ref):
    o_ref[...] = p_ref[0] + p_ref[1]


def _final_add(rstp):
    return pl.pallas_call(
        _add_body,
        grid=(N // MMB,),
        in_specs=[pl.BlockSpec((NC, MMB, D), lambda i: (0, i, 0))],
        out_specs=pl.BlockSpec((MMB, D), lambda i: (i, 0)),
        out_shape=jax.ShapeDtypeStruct((N, D), jnp.float32),
    )(rstp)


def kernel(feat, edge_index, W_src, b_src, W_dst, b_dst, attn):
    ei32 = edge_index.astype(jnp.int32)
    ei = jnp.stack([ei32[0].reshape(-1, C), ei32[1].reshape(-1, C)],
                   axis=1).reshape(-1)
    fs, fd = _mm(feat, W_src, b_src.reshape(1, D), W_dst, b_dst.reshape(1, D))
    a06 = 0.6 * attn
    a04 = 0.4 * attn
    expsc, denp = _scores(fs, fd, ei, a06, a04)
    den = _den_merge(denp)
    rstp = _agg(fs, ei, expsc, den)
    return _final_add(rstp)
